# Initial kernel scaffold; baseline (speedup 1.0000x reference)
import jax
import jax.numpy as jnp
from jax.experimental import pallas as pl


def kernel(prefix, table, W1, b1, W2, b2):
    raise NotImplementedError("write your pallas kernel here")


# speedup vs baseline: 1.0900x; 1.0900x over previous
import jax
import jax.numpy as jnp
from jax.experimental import pallas as pl
from jax.experimental.pallas import tpu as pltpu

PRE_SEQ_LEN = 64
HIDDEN = 1024
OUT_DIM = 2 * 24 * 1024
BATCH = 8
ROWS = BATCH * PRE_SEQ_LEN
BN = 2048


def _body(pf_ref, table_ref, W1_ref, b1_ref, W2_ref, b2_ref, out_ref,
          h_ref, p_ref):
    j = pl.program_id(0)

    @pl.when(j == 0)
    def _():
        emb = table_ref[...]
        h = jnp.tanh(
            jnp.dot(emb, W1_ref[...], preferred_element_type=jnp.float32)
            + b1_ref[...])
        h_ref[...] = h
        pf = pf_ref[...]
        iota = jax.lax.broadcasted_iota(jnp.int32, (ROWS, PRE_SEQ_LEN), 1)
        p_ref[...] = (pf == iota).astype(jnp.float32)

    ob = (jnp.dot(h_ref[...], W2_ref[...], preferred_element_type=jnp.float32)
          + b2_ref[...])
    out_ref[...] = jnp.dot(p_ref[...], ob, preferred_element_type=jnp.float32)


def kernel(prefix, table, W1, b1, W2, b2):
    pf2d = prefix.reshape(ROWS, 1).astype(jnp.int32)
    b1r = b1.reshape(1, HIDDEN)
    b2r = b2.reshape(1, OUT_DIM)
    grid = (OUT_DIM // BN,)
    out = pl.pallas_call(
        _body,
        grid=grid,
        in_specs=[
            pl.BlockSpec((ROWS, 1), lambda j: (0, 0)),
            pl.BlockSpec((PRE_SEQ_LEN, HIDDEN), lambda j: (0, 0)),
            pl.BlockSpec((HIDDEN, HIDDEN), lambda j: (0, 0)),
            pl.BlockSpec((1, HIDDEN), lambda j: (0, 0)),
            pl.BlockSpec((HIDDEN, BN), lambda j: (0, j)),
            pl.BlockSpec((1, BN), lambda j: (0, j)),
        ],
        out_specs=pl.BlockSpec((ROWS, BN), lambda j: (0, j)),
        out_shape=jax.ShapeDtypeStruct((ROWS, OUT_DIM), jnp.float32),
        scratch_shapes=[
            pltpu.VMEM((PRE_SEQ_LEN, HIDDEN), jnp.float32),
            pltpu.VMEM((ROWS, PRE_SEQ_LEN), jnp.float32),
        ],
    )(pf2d, table, W1, b1r, W2, b2r)
    return out.reshape(BATCH, PRE_SEQ_LEN, OUT_DIM)
